# baseline (speedup 1.0000x reference)
import functools

import jax
import jax.numpy as jnp
from jax import lax
from jax.experimental import pallas as pl
from jax.experimental.pallas import tpu as pltpu
from jax.experimental.pallas import tpu_sc as plsc

B, L, V, D, H, C = 4096, 200, 100000, 128, 1024, 3

NC, NS, LANES = 2, 16, 16
NW = NC * NS
HALF = L // 2
DCH = D // LANES

NCHUNK = 1
CB = B // NCHUNK
SPW = CB // NW
NBUF = 3


def _pool_body(sent_hbm, table_hbm, out_hbm, idx_v, rows_v, out_v, sem0, sem1,
               sem2):
    wid = lax.axis_index("s") * NC + lax.axis_index("c")
    base = wid * SPW
    pltpu.sync_copy(sent_hbm.at[pl.ds(base, SPW)], idx_v)

    sems = (sem0, sem1, sem2)

    def fire(j, b):
        for k in range(2):
            for q in range(2):
                pltpu.async_copy(
                    table_hbm.at[idx_v.at[j, k, pl.ds(50 * q, 50)]],
                    rows_v.at[b, k, pl.ds(50 * q, 50)], sems[b])

    def drain(b):
        for k in range(2):
            pltpu.make_async_copy(table_hbm.at[idx_v.at[0, k]],
                                  rows_v.at[b, k], sems[b]).wait()

    def reduce_into(j, b):
        def rbody(r, accs):
            new = []
            for c in range(DCH):
                v = accs[c]
                v = v + rows_v[b, 0, r, pl.ds(c * LANES, LANES)]
                v = v + rows_v[b, 1, r, pl.ds(c * LANES, LANES)]
                new.append(v)
            return tuple(new)

        accs = tuple(jnp.zeros((LANES,), jnp.float32) for _ in range(DCH))
        accs = lax.fori_loop(0, HALF, rbody, accs, unroll=5)
        for c in range(DCH):
            out_v[j, pl.ds(c * LANES, LANES)] = accs[c]

    for b in range(NBUF):
        fire(b, b)

    def loop_body(p, _):
        jj = p * NBUF
        for b in range(NBUF):
            j = jj + b
            drain(b)
            reduce_into(j, b)

            @pl.when(j + NBUF < SPW)
            def _():
                fire(j + NBUF, b)
        return _

    lax.fori_loop(0, SPW // NBUF, loop_body, None)
    for i, j in enumerate(range(NBUF * (SPW // NBUF), SPW)):
        drain(i)
        reduce_into(j, i)
    pltpu.sync_copy(out_v, out_hbm.at[pl.ds(base, SPW)])


@functools.partial(
    pl.kernel,
    out_type=jax.ShapeDtypeStruct((CB, D), jnp.float32),
    mesh=plsc.VectorSubcoreMesh(core_axis_name="c", subcore_axis_name="s",
                                num_cores=NC, num_subcores=NS),
    scratch_types=[
        pltpu.VMEM((SPW, 2, HALF), jnp.int32),
        pltpu.VMEM((NBUF, 2, HALF, D), jnp.float32),
        pltpu.VMEM((SPW, D), jnp.float32),
        pltpu.SemaphoreType.DMA,
        pltpu.SemaphoreType.DMA,
        pltpu.SemaphoreType.DMA,
    ],
    name="sc_embed_pool",
)
def _pool(sent_hbm, table_hbm, out_hbm, idx_v, rows_v, out_v, sem0, sem1,
          sem2):
    _pool_body(sent_hbm, table_hbm, out_hbm, idx_v, rows_v, out_v, sem0, sem1,
               sem2)


def _mlp_body(x_ref, w1_ref, b1_ref, w2_ref, b2_ref, w3_ref, b3_ref, o_ref):
    bf = jnp.bfloat16
    z = jnp.dot(x_ref[...].astype(bf), w1_ref[...].astype(bf),
                preferred_element_type=jnp.float32)
    z = jnp.maximum(z + b1_ref[...], 0.0)
    z = jnp.dot(z.astype(bf), w2_ref[...].astype(bf),
                preferred_element_type=jnp.float32)
    z = jnp.maximum(z + b2_ref[...], 0.0)
    o_ref[...] = (jnp.dot(z.astype(bf), w3_ref[...].astype(bf),
                          preferred_element_type=jnp.float32) + b3_ref[...])


def _mlp(pooled, W1, b1, W2, b2, W3p, b3p):
    BM = 512
    return pl.pallas_call(
        _mlp_body,
        grid=(CB // BM,),
        in_specs=[
            pl.BlockSpec((BM, D), lambda i: (i, 0)),
            pl.BlockSpec((D, H), lambda i: (0, 0)),
            pl.BlockSpec((1, H), lambda i: (0, 0)),
            pl.BlockSpec((H, H), lambda i: (0, 0)),
            pl.BlockSpec((1, H), lambda i: (0, 0)),
            pl.BlockSpec((H, 128), lambda i: (0, 0)),
            pl.BlockSpec((1, 128), lambda i: (0, 0)),
        ],
        out_specs=pl.BlockSpec((BM, 128), lambda i: (i, 0)),
        out_shape=jax.ShapeDtypeStruct((CB, 128), jnp.float32),
    )(pooled, W1, b1, W2, b2, W3p, b3p)


def kernel(sentences, transitions, table, W1, b1, W2, b2, W3, b3):
    del transitions
    sent = sentences.reshape(NCHUNK, CB, 2, HALF)
    W3p = jnp.zeros((H, 128), W3.dtype).at[:, :C].set(W3)
    b3p = jnp.zeros((128,), b3.dtype).at[:C].set(b3)
    b1r, b2r, b3r = b1.reshape(1, H), b2.reshape(1, H), b3p.reshape(1, 128)
    outs = []
    for k in range(NCHUNK):
        pooled = _pool(sent[k], table)
        outs.append(_mlp(pooled, W1, b1r, W2, b2r, W3p, b3r))
    return jnp.concatenate(outs, axis=0)[:, :C]


# speedup vs baseline: 1.0075x; 1.0075x over previous
import functools

import jax
import jax.numpy as jnp
from jax import lax
from jax.experimental import pallas as pl
from jax.experimental.pallas import tpu as pltpu
from jax.experimental.pallas import tpu_sc as plsc

B, L, V, D, H, C = 4096, 200, 100000, 128, 1024, 3

NC, NS, LANES = 2, 16, 16
NW = NC * NS
HALF = L // 2
DCH = D // LANES

NCHUNK = 1
CB = B // NCHUNK
SPW = CB // NW
NUNIT = 6


def _pool_body(sent_hbm, table_hbm, out_hbm, idx_v, rows_v, out_v, sem0, sem1,
               sem2, sem3, sem4, sem5):
    wid = lax.axis_index("s") * NC + lax.axis_index("c")
    base = wid * SPW
    pltpu.sync_copy(sent_hbm.at[pl.ds(base, SPW)], idx_v)

    sems = (sem0, sem1, sem2, sem3, sem4, sem5)

    def fire(j, k, b):
        pltpu.async_copy(table_hbm.at[idx_v.at[j, k]], rows_v.at[b],
                         sems[b])

    def drain(b):
        pltpu.make_async_copy(table_hbm.at[idx_v.at[0, 0]],
                              rows_v.at[b], sems[b]).wait()

    def reduce_half(b, accs):
        def rbody(r, a):
            new = []
            for c in range(DCH):
                new.append(a[c] + rows_v[b, r, pl.ds(c * LANES, LANES)])
            return tuple(new)

        return lax.fori_loop(0, HALF, rbody, accs, unroll=5)

    def store_accs(j, accs):
        for c in range(DCH):
            out_v[j, pl.ds(c * LANES, LANES)] = accs[c]

    for b in range(NUNIT):
        fire(b // 2, b % 2, b)

    zeros = tuple(jnp.zeros((LANES,), jnp.float32) for _ in range(DCH))

    def do_sentence(j, m):
        drain(2 * m)
        accs = reduce_half(2 * m, zeros)

        @pl.when(j + 3 < SPW)
        def _():
            fire(j + 3, 0, 2 * m)

        drain(2 * m + 1)
        accs = reduce_half(2 * m + 1, accs)

        @pl.when(j + 3 < SPW)
        def _():
            fire(j + 3, 1, 2 * m + 1)

        store_accs(j, accs)

    def loop_body(p, _):
        for m in range(3):
            do_sentence(3 * p + m, m)
        return _

    lax.fori_loop(0, SPW // 3, loop_body, None)
    for m, j in enumerate(range(3 * (SPW // 3), SPW)):
        do_sentence(j, m)
    pltpu.sync_copy(out_v, out_hbm.at[pl.ds(base, SPW)])


@functools.partial(
    pl.kernel,
    out_type=jax.ShapeDtypeStruct((CB, D), jnp.float32),
    mesh=plsc.VectorSubcoreMesh(core_axis_name="c", subcore_axis_name="s",
                                num_cores=NC, num_subcores=NS),
    scratch_types=[
        pltpu.VMEM((SPW, 2, HALF), jnp.int32),
        pltpu.VMEM((NUNIT, HALF, D), jnp.float32),
        pltpu.VMEM((SPW, D), jnp.float32),
    ] + [pltpu.SemaphoreType.DMA] * 6,
    name="sc_embed_pool",
)
def _pool(sent_hbm, table_hbm, out_hbm, idx_v, rows_v, out_v, sem0, sem1,
          sem2, sem3, sem4, sem5):
    _pool_body(sent_hbm, table_hbm, out_hbm, idx_v, rows_v, out_v, sem0, sem1,
               sem2, sem3, sem4, sem5)


def _mlp_body(x_ref, w1_ref, b1_ref, w2_ref, b2_ref, w3_ref, b3_ref, o_ref):
    bf = jnp.bfloat16
    z = jnp.dot(x_ref[...].astype(bf), w1_ref[...].astype(bf),
                preferred_element_type=jnp.float32)
    z = jnp.maximum(z + b1_ref[...], 0.0)
    z = jnp.dot(z.astype(bf), w2_ref[...].astype(bf),
                preferred_element_type=jnp.float32)
    z = jnp.maximum(z + b2_ref[...], 0.0)
    o_ref[...] = (jnp.dot(z.astype(bf), w3_ref[...].astype(bf),
                          preferred_element_type=jnp.float32) + b3_ref[...])


def _mlp(pooled, W1, b1, W2, b2, W3p, b3p):
    BM = 512
    return pl.pallas_call(
        _mlp_body,
        grid=(CB // BM,),
        in_specs=[
            pl.BlockSpec((BM, D), lambda i: (i, 0)),
            pl.BlockSpec((D, H), lambda i: (0, 0)),
            pl.BlockSpec((1, H), lambda i: (0, 0)),
            pl.BlockSpec((H, H), lambda i: (0, 0)),
            pl.BlockSpec((1, H), lambda i: (0, 0)),
            pl.BlockSpec((H, 128), lambda i: (0, 0)),
            pl.BlockSpec((1, 128), lambda i: (0, 0)),
        ],
        out_specs=pl.BlockSpec((BM, 128), lambda i: (i, 0)),
        out_shape=jax.ShapeDtypeStruct((CB, 128), jnp.float32),
    )(pooled, W1, b1, W2, b2, W3p, b3p)


def kernel(sentences, transitions, table, W1, b1, W2, b2, W3, b3):
    del transitions
    sent = sentences.reshape(NCHUNK, CB, 2, HALF)
    W3p = jnp.zeros((H, 128), W3.dtype).at[:, :C].set(W3)
    b3p = jnp.zeros((128,), b3.dtype).at[:C].set(b3)
    b1r, b2r, b3r = b1.reshape(1, H), b2.reshape(1, H), b3p.reshape(1, 128)
    outs = []
    for k in range(NCHUNK):
        pooled = _pool(sent[k], table)
        outs.append(_mlp(pooled, W1, b1r, W2, b2r, W3p, b3r))
    return jnp.concatenate(outs, axis=0)[:, :C]


# speedup vs baseline: 1.0109x; 1.0034x over previous
import functools

import jax
import jax.numpy as jnp
from jax import lax
from jax.experimental import pallas as pl
from jax.experimental.pallas import tpu as pltpu
from jax.experimental.pallas import tpu_sc as plsc

B, L, V, D, H, C = 4096, 200, 100000, 128, 1024, 3

NC, NS, LANES = 2, 16, 16
NW = NC * NS
HALF = L // 2
DCH = D // LANES

NCHUNK = 1
CB = B // NCHUNK
SPW = CB // NW
NUNIT = 6
CP = 8


def _pool_body(sent_hbm, table_hbm, out_hbm, idx_v, rows_v, out_v, sem0, sem1,
               sem2, sem3, sem4, sem5):
    wid = lax.axis_index("s") * NC + lax.axis_index("c")
    base = wid * SPW
    pltpu.sync_copy(sent_hbm.at[pl.ds(base, SPW)], idx_v)

    sems = (sem0, sem1, sem2, sem3, sem4, sem5)

    def fire(j, k, b):
        pltpu.async_copy(table_hbm.at[idx_v.at[j, k]], rows_v.at[b],
                         sems[b])

    def drain(b):
        pltpu.make_async_copy(table_hbm.at[idx_v.at[0, 0]],
                              rows_v.at[b], sems[b]).wait()

    def reduce_half(b, accs):
        def rbody(r, a):
            new = []
            for c in range(DCH):
                new.append(a[c] + rows_v[b, r, pl.ds(c * LANES, LANES)])
            return tuple(new)

        return lax.fori_loop(0, HALF, rbody, accs, unroll=5)

    def store_accs(j, accs):
        for c in range(DCH):
            out_v[j, pl.ds(c * LANES, LANES)] = accs[c]

    for b in range(NUNIT):
        fire(b // 2, b % 2, b)

    zeros = tuple(jnp.zeros((LANES,), jnp.float32) for _ in range(DCH))

    def do_sentence(j, m):
        drain(2 * m)
        accs = reduce_half(2 * m, zeros)

        @pl.when(j + 3 < SPW)
        def _():
            fire(j + 3, 0, 2 * m)

        drain(2 * m + 1)
        accs = reduce_half(2 * m + 1, accs)

        @pl.when(j + 3 < SPW)
        def _():
            fire(j + 3, 1, 2 * m + 1)

        store_accs(j, accs)

    def loop_body(p, _):
        for m in range(3):
            do_sentence(3 * p + m, m)
        return _

    lax.fori_loop(0, SPW // 3, loop_body, None)
    for m, j in enumerate(range(3 * (SPW // 3), SPW)):
        do_sentence(j, m)
    pltpu.sync_copy(out_v, out_hbm.at[pl.ds(base, SPW)])


@functools.partial(
    pl.kernel,
    out_type=jax.ShapeDtypeStruct((CB, D), jnp.float32),
    mesh=plsc.VectorSubcoreMesh(core_axis_name="c", subcore_axis_name="s",
                                num_cores=NC, num_subcores=NS),
    scratch_types=[
        pltpu.VMEM((SPW, 2, HALF), jnp.int32),
        pltpu.VMEM((NUNIT, HALF, D), jnp.float32),
        pltpu.VMEM((SPW, D), jnp.float32),
    ] + [pltpu.SemaphoreType.DMA] * 6,
    name="sc_embed_pool",
)
def _pool(sent_hbm, table_hbm, out_hbm, idx_v, rows_v, out_v, sem0, sem1,
          sem2, sem3, sem4, sem5):
    _pool_body(sent_hbm, table_hbm, out_hbm, idx_v, rows_v, out_v, sem0, sem1,
               sem2, sem3, sem4, sem5)


def _mlp_body(x_ref, w1_ref, b1_ref, w2_ref, b2_ref, w3_ref, b3_ref, o_ref):
    bf = jnp.bfloat16
    z = jnp.dot(x_ref[...].astype(bf), w1_ref[...],
                preferred_element_type=jnp.float32)
    z = jnp.maximum(z + b1_ref[...], 0.0)
    z = jnp.dot(z.astype(bf), w2_ref[...], preferred_element_type=jnp.float32)
    z = jnp.maximum(z + b2_ref[...], 0.0)
    o_ref[...] = (jnp.dot(z.astype(bf), w3_ref[...],
                          preferred_element_type=jnp.float32) + b3_ref[...])


def _mlp(pooled, W1, b1, W2, b2, W3p, b3p):
    BM = 512
    return pl.pallas_call(
        _mlp_body,
        grid=(CB // BM,),
        in_specs=[
            pl.BlockSpec((BM, D), lambda i: (i, 0)),
            pl.BlockSpec((D, H), lambda i: (0, 0)),
            pl.BlockSpec((1, H), lambda i: (0, 0)),
            pl.BlockSpec((H, H), lambda i: (0, 0)),
            pl.BlockSpec((1, H), lambda i: (0, 0)),
            pl.BlockSpec((H, CP), lambda i: (0, 0)),
            pl.BlockSpec((1, CP), lambda i: (0, 0)),
        ],
        out_specs=pl.BlockSpec((BM, CP), lambda i: (i, 0)),
        out_shape=jax.ShapeDtypeStruct((CB, CP), jnp.float32),
    )(pooled, W1, b1, W2, b2, W3p, b3p)


def kernel(sentences, transitions, table, W1, b1, W2, b2, W3, b3):
    del transitions
    bf = jnp.bfloat16
    sent = sentences.reshape(NCHUNK, CB, 2, HALF)
    W3p = jnp.zeros((H, CP), bf).at[:, :C].set(W3.astype(bf))
    b3p = jnp.zeros((CP,), b3.dtype).at[:C].set(b3)
    b1r, b2r, b3r = b1.reshape(1, H), b2.reshape(1, H), b3p.reshape(1, CP)
    W1b, W2b = W1.astype(bf), W2.astype(bf)
    outs = []
    for k in range(NCHUNK):
        pooled = _pool(sent[k], table)
        outs.append(_mlp(pooled, W1b, b1r, W2b, b2r, W3p, b3r))
    return jnp.concatenate(outs, axis=0)[:, :C]
